# baseline (speedup 1.0000x reference)
import functools

import jax
import jax.numpy as jnp
from jax import lax
from jax.experimental import pallas as pl
from jax.experimental.pallas import tpu as pltpu
from jax.experimental.pallas import tpu_sc as plsc

VOCAB = 1000000
D = 64
BATCH = 4096
SEQ = 200
PAD = 0

NC, NS, L = 2, 16, 16
NW = NC * NS

VCHUNK = 8192
VGRID = (VOCAB + VCHUNK - 1) // VCHUNK


def _table_body(x_ref, o_ref):
    o_ref[:, 0:D] = jnp.transpose(x_ref[...])
    o_ref[:, D:128] = jnp.zeros((VCHUNK, 128 - D), jnp.float32)


def _pad_table(table_t):
    return pl.pallas_call(
        _table_body,
        grid=(VGRID,),
        in_specs=[pl.BlockSpec((D, VCHUNK), lambda i: (0, i))],
        out_specs=pl.BlockSpec((VCHUNK, 128), lambda i: (i, 0)),
        out_shape=jax.ShapeDtypeStruct((VOCAB, 128), jnp.float32),
    )(table_t)


def _out_body(x_ref, i_ref, o_ref):
    x = x_ref[0, :, 0:D]
    m = (i_ref[0] != PAD).astype(jnp.float32)
    o_ref[0] = jnp.transpose(x) * m


def _mask_transpose(rows, idx_t):
    return pl.pallas_call(
        _out_body,
        grid=(SEQ,),
        in_specs=[
            pl.BlockSpec((1, BATCH, 128), lambda i: (i, 0, 0)),
            pl.BlockSpec((1, 1, BATCH), lambda i: (i, 0, 0)),
        ],
        out_specs=pl.BlockSpec((1, D, BATCH), lambda i: (i, 0, 0)),
        out_shape=jax.ShapeDtypeStruct((SEQ, D, BATCH), jnp.float32),
    )(rows, idx_t.reshape(SEQ, 1, BATCH))


@functools.partial(
    pl.kernel,
    out_type=jax.ShapeDtypeStruct((SEQ, BATCH, 128), jnp.float32),
    mesh=plsc.VectorSubcoreMesh(core_axis_name="c", subcore_axis_name="s"),
    scratch_types=[
        pltpu.VMEM((3, 128), jnp.int32),
        pltpu.VMEM((2, 128, 128), jnp.float32),
        pltpu.SemaphoreType.DMA,
        pltpu.SemaphoreType.DMA,
        pltpu.SemaphoreType.DMA,
    ],
    compiler_params=pltpu.CompilerParams(
        needs_layout_passes=False, use_tc_tiling_on_sc=True
    ),
)
def _emb_gather(idx_hbm, tbl_hbm, out_hbm, idx_r, rows_r, isem, gsem, osem):
    wid = lax.axis_index("s") * NC + lax.axis_index("c")
    b0 = wid * 128

    def idx_start(j):
        pltpu.async_copy(idx_hbm.at[j, pl.ds(b0, 128)], idx_r.at[j % 3], isem)

    def idx_wait(j):
        pltpu.make_async_copy(
            idx_hbm.at[j, pl.ds(b0, 128)], idx_r.at[j % 3], isem
        ).wait()

    def gather_start(j, b):
        pltpu.async_copy(tbl_hbm.at[idx_r.at[j % 3]], rows_r.at[b], gsem)

    def gather_wait(b):
        pltpu.make_async_copy(tbl_hbm.at[idx_r.at[0]], rows_r.at[b], gsem).wait()

    def out_start(j, b):
        pltpu.async_copy(
            rows_r.at[b], out_hbm.at[j, pl.ds(b0, 128), :], osem
        )

    def out_wait(j, b):
        pltpu.make_async_copy(
            rows_r.at[b], out_hbm.at[j, pl.ds(b0, 128), :], osem
        ).wait()

    idx_start(0)
    idx_start(1)
    idx_wait(0)
    gather_start(0, 0)

    def half(i, b, carry):
        j = i * 2 + b

        @pl.when(j < SEQ - 1)
        def _():
            idx_wait(j + 1)

            @pl.when(j >= 1)
            def _():
                out_wait(j - 1, 1 - b)

            gather_start(j + 1, 1 - b)

        @pl.when(j < SEQ - 2)
        def _():
            idx_start(j + 2)

        gather_wait(b)
        out_start(j, b)
        return carry

    def pair(i, carry):
        half(i, 0, carry)
        half(i, 1, carry)
        return carry

    lax.fori_loop(0, SEQ // 2, pair, 0)
    out_wait(SEQ - 1, 1)


def kernel(inputs, embedding_matrix):
    idx_t = jnp.transpose(inputs).astype(jnp.int32)
    tbl = _pad_table(jnp.transpose(embedding_matrix))
    rows = _emb_gather(idx_t, tbl)
    out = _mask_transpose(rows, idx_t)
    return jnp.transpose(out, (2, 0, 1))
